# baseline (speedup 1.0000x reference)
import functools

import jax
import jax.numpy as jnp
from jax import lax
from jax.experimental import pallas as pl
from jax.experimental.pallas import tpu as pltpu
from jax.experimental.pallas import tpu_sc as plsc

N = 4
V = 2500
VP = 2560
E = 80000
D_IN = 16
DM = 128
NW = 32
K = 40
PASSES = VP // (K * NW)
CHUNK = 4000
CH2 = 2 * CHUNK
NCHPG = E // CHUNK
EPS = 1e-5


def _adj_body(comb_hbm, out_hbm, buf, acc, sidx, sval, esem, osem):
    cid = lax.axis_index("c")
    sid = lax.axis_index("s")
    wid = sid * 2 + cid

    zero16 = jnp.zeros((16,), jnp.float32)

    def zero_body(i, _):
        acc[pl.ds(i * 16, 16)] = zero16
        return 0

    def edge_dma(g, ch, off):
        base = (g * NCHPG + ch) * CH2
        return pltpu.make_async_copy(comb_hbm.at[pl.ds(base, CH2)],
                                     buf.at[pl.ds(off, CH2)], esem)

    slots = [(g, p) for g in range(N) for p in range(PASSES)]
    edge_dma(0, 0, 0).start()
    wo = None
    for s, (g, p) in enumerate(slots):
        c0 = (p * NW + wid) * (K * VP)
        next_g = slots[s + 1][0] if s + 1 < len(slots) else None

        if wo is not None:
            wo.wait()
        lax.fori_loop(0, (K * VP) // 16, zero_body, 0)

        def chunk_body(ch, _, g=g, c0=c0, next_g=next_g):
            off = (ch % 2) * CH2
            edge_dma(g, ch, off).wait()

            @pl.when(ch < NCHPG - 1)
            def _():
                edge_dma(g, ch + 1, CH2 - off).start()

            if next_g is not None:
                @pl.when(ch == NCHPG - 1)
                def _():
                    edge_dma(next_g, 0, CH2 - off).start()

            @plsc.parallel_loop(0, CHUNK // 16, unroll=4,
                                carry=jnp.int32(0))
            def scan_body(j, cnt):
                i16 = buf[pl.ds(off + j * 16, 16)]
                w16 = buf[pl.ds(off + CHUNK + j * 16, 16)]
                rel = i16 - c0
                m = plsc.bitcast(rel, jnp.uint32) < jnp.uint32(K * VP)
                plsc.store_compressed(sidx.at[pl.ds(cnt, 16)], rel, mask=m)
                plsc.store_compressed(sval.at[pl.ds(cnt, 16)], w16, mask=m)
                return cnt + plsc.all_reduce_population_count(m)[0]

            sidx[pl.ds(scan_body, 16)] = jnp.zeros((16,), jnp.int32)
            sval[pl.ds(scan_body, 16)] = jnp.zeros((16,), jnp.int32)

            def drain_body(t, _):
                relc = sidx[pl.ds(t * 16, 16)]
                w = plsc.bitcast(sval[pl.ds(t * 16, 16)], jnp.float32)
                plsc.addupdate_scatter(acc, [relc], w)
                return 0

            lax.fori_loop(0, (scan_body + 15) // 16, drain_body, 0)
            return 0

        lax.fori_loop(0, NCHPG, chunk_body, 0)

        wo = pltpu.make_async_copy(
            acc, out_hbm.at[pl.ds(g * VP * VP + c0, K * VP)], osem)
        wo.start()
    wo.wait()


def _build_dense_adj(comb):
    kern = pl.kernel(
        _adj_body,
        out_type=jax.ShapeDtypeStruct((N * VP * VP,), jnp.float32),
        mesh=plsc.VectorSubcoreMesh(core_axis_name="c", subcore_axis_name="s",
                                    num_cores=2, num_subcores=16),
        compiler_params=pltpu.CompilerParams(needs_layout_passes=False),
        scratch_types=[
            pltpu.VMEM((2 * CH2,), jnp.int32),
            pltpu.VMEM((K * VP,), jnp.float32),
            pltpu.VMEM((CHUNK + 16,), jnp.int32),
            pltpu.VMEM((CHUNK + 16,), jnp.int32),
            pltpu.SemaphoreType.DMA,
            pltpu.SemaphoreType.DMA,
        ],
    )
    return kern(comb)


def _layer_norm(h, g, b):
    mu = jnp.mean(h, axis=-1, keepdims=True)
    var = jnp.mean((h - mu) ** 2, axis=-1, keepdims=True)
    return (h - mu) * lax.rsqrt(var + EPS) * g + b


RB = 512
NRB = VP // RB
PREC = lax.Precision.HIGHEST


def _fused_body(a_ref, x_ref, w_ref, b_ref, lg_ref, lb_ref, hw_ref, hb_ref,
                o_ref, abf_scr, h_scr, v_scr, dinv_scr, pool_scr):
    g = pl.program_id(0)
    l = pl.program_id(1)
    rb = pl.program_id(2)
    srow = g * VP + rb * RB
    lg = lg_ref[...]
    lb = lb_ref[...]

    @pl.when(l == 0)
    def _():
        a = a_ref[...]
        s = jnp.sum(a, axis=1, keepdims=True)
        dinv_scr[pl.ds(srow, RB), :] = lax.rsqrt(s + 1.0)
        abf_scr[pl.ds(rb * RB, RB), :] = a.astype(jnp.bfloat16)
        h_scr[pl.ds(srow, RB), :] = x_ref[pl.ds(rb * RB, RB), :]

    @pl.when((l > 0) & (rb == 0))
    def _():
        hg = h_scr[pl.ds(g * VP, VP), :]
        u = jnp.dot(hg, w_ref[0], preferred_element_type=jnp.float32,
                    precision=PREC)
        v_scr[...] = (dinv_scr[pl.ds(g * VP, VP), :] * u).astype(jnp.bfloat16)

    @pl.when(l > 0)
    def _():
        y = jnp.dot(abf_scr[pl.ds(rb * RB, RB), :], v_scr[...],
                    preferred_element_type=jnp.float32)
        y = y + v_scr[pl.ds(rb * RB, RB), :].astype(jnp.float32)
        h = dinv_scr[pl.ds(srow, RB), :] * y + b_ref[0, 0]
        h = jnp.maximum(h, 0.0)
        h = _layer_norm(h, lg, lb)
        h_scr[pl.ds(srow, RB), :] = h

        @pl.when(l == 3)
        def _():
            rowid = lax.broadcasted_iota(jnp.int32, (RB, 1), 0) + rb * RB
            hm = jnp.where(rowid < V, h, -1e30)
            m = jnp.max(hm, axis=0, keepdims=True)

            @pl.when(rb == 0)
            def _():
                pool_scr[pl.ds(g, 1), :] = m

            @pl.when(rb > 0)
            def _():
                pool_scr[pl.ds(g, 1), :] = jnp.maximum(
                    pool_scr[pl.ds(g, 1), :], m)

            @pl.when((g == N - 1) & (rb == NRB - 1))
            def _():
                pn = _layer_norm(pool_scr[...], lg, lb)
                o_ref[...] = jnp.dot(
                    pn, hw_ref[...], preferred_element_type=jnp.float32,
                    precision=PREC) + hb_ref[...]


def _run_dense(adj, x, W0, b0, W1, b1, W2, b2, ln_g, ln_b, head_W, head_b):
    xp = jnp.pad(x.reshape(N, V, D_IN),
                 ((0, 0), (0, VP - V), (0, DM - D_IN))).reshape(N * VP, DM)
    ws = jnp.stack([jnp.pad(W0, ((0, DM - D_IN), (0, 0))), W1, W2])
    bs = jnp.stack([b0, b1, b2]).reshape(3, 1, DM)

    full = lambda shape: pl.BlockSpec(shape, lambda g, l, r: (0,) * len(shape))
    return pl.pallas_call(
        _fused_body,
        grid=(N, 4, NRB),
        in_specs=[
            pl.BlockSpec((RB, VP),
                         lambda g, l, r: (g * NRB + jnp.where(l == 0, r, 0), 0)),
            pl.BlockSpec((VP, DM), lambda g, l, r: (g, 0)),
            pl.BlockSpec((1, DM, DM),
                         lambda g, l, r: (jnp.maximum(l, 1) - 1, 0, 0)),
            pl.BlockSpec((1, 1, DM),
                         lambda g, l, r: (jnp.maximum(l, 1) - 1, 0, 0)),
            full((DM,)), full((DM,)),
            full((DM, 3)), full((3,)),
        ],
        out_specs=pl.BlockSpec((N, 3), lambda g, l, r: (0, 0)),
        out_shape=jax.ShapeDtypeStruct((N, 3), jnp.float32),
        scratch_shapes=[
            pltpu.VMEM((VP, VP), jnp.bfloat16),
            pltpu.VMEM((N * VP, DM), jnp.float32),
            pltpu.VMEM((VP, DM), jnp.bfloat16),
            pltpu.VMEM((N * VP, 1), jnp.float32),
            pltpu.VMEM((N, DM), jnp.float32),
        ],
    )(adj, xp, ws, bs, ln_g, ln_b, head_W, head_b)


def kernel(x, edge_index, edge_weight, W0, b0, W1, b1, W2, b2,
           ln_g, ln_b, head_W, head_b):
    ei = edge_index.reshape(2, -1).astype(jnp.int32)
    flat_idx = ei[1] * VP + ei[0]
    ews = edge_weight.reshape(-1).astype(jnp.float32)
    comb = jnp.concatenate(
        [flat_idx.reshape(N * NCHPG, CHUNK),
         lax.bitcast_convert_type(ews, jnp.int32).reshape(N * NCHPG, CHUNK)],
        axis=1).reshape(-1)

    adj_flat = _build_dense_adj(comb)
    adj = adj_flat.reshape(N * VP, VP)

    return _run_dense(adj, x, W0, b0, W1, b1, W2, b2,
                      ln_g, ln_b, head_W, head_b)


# speedup vs baseline: 1.0020x; 1.0020x over previous
import functools

import jax
import jax.numpy as jnp
from jax import lax
from jax.experimental import pallas as pl
from jax.experimental.pallas import tpu as pltpu
from jax.experimental.pallas import tpu_sc as plsc

N = 4
V = 2500
VP = 2560
E = 80000
D_IN = 16
DM = 128
NW = 32
K = 40
PASSES = VP // (K * NW)
CHUNK = 4000
CH2 = 2 * CHUNK
NCHPG = E // CHUNK
EPS = 1e-5


def _adj_body(comb_hbm, out_hbm, buf, acc, esem, osem):
    cid = lax.axis_index("c")
    sid = lax.axis_index("s")
    wid = sid * 2 + cid

    zero16 = jnp.zeros((16,), jnp.float32)

    def zero_body(i, _):
        acc[pl.ds(i * 16, 16)] = zero16
        return 0

    def edge_dma(g, ch, off):
        base = (g * NCHPG + ch) * CH2
        return pltpu.make_async_copy(comb_hbm.at[pl.ds(base, CH2)],
                                     buf.at[pl.ds(off, CH2)], esem)

    slots = [(g, p) for g in range(N) for p in range(PASSES)]
    edge_dma(0, 0, 0).start()
    wo = None
    for s, (g, p) in enumerate(slots):
        c0 = (p * NW + wid) * (K * VP)
        next_g = slots[s + 1][0] if s + 1 < len(slots) else None

        if wo is not None:
            wo.wait()
        lax.fori_loop(0, (K * VP) // 16, zero_body, 0)

        def chunk_body(ch, _, g=g, c0=c0, next_g=next_g):
            off = (ch % 2) * CH2
            edge_dma(g, ch, off).wait()

            @pl.when(ch < NCHPG - 1)
            def _():
                edge_dma(g, ch + 1, CH2 - off).start()

            if next_g is not None:
                @pl.when(ch == NCHPG - 1)
                def _():
                    edge_dma(next_g, 0, CH2 - off).start()

            @plsc.parallel_loop(0, CHUNK // 64, unroll=2)
            def edge_body(j):
                base = off + j * 64
                wbase = base + CHUNK
                for t in range(4):
                    i16 = buf[pl.ds(base + t * 16, 16)]
                    w16 = buf[pl.ds(wbase + t * 16, 16)]
                    rel = i16 - c0
                    m = plsc.bitcast(rel, jnp.uint32) < jnp.uint32(K * VP)
                    relc = jnp.where(m, rel, 0)
                    plsc.addupdate_scatter(acc, [relc],
                                           plsc.bitcast(w16, jnp.float32),
                                           mask=m)

            return 0

        lax.fori_loop(0, NCHPG, chunk_body, 0)

        wo = pltpu.make_async_copy(
            acc, out_hbm.at[pl.ds(g * VP * VP + c0, K * VP)], osem)
        wo.start()
    wo.wait()


def _build_dense_adj(comb):
    kern = pl.kernel(
        _adj_body,
        out_type=jax.ShapeDtypeStruct((N * VP * VP,), jnp.float32),
        mesh=plsc.VectorSubcoreMesh(core_axis_name="c", subcore_axis_name="s",
                                    num_cores=2, num_subcores=16),
        compiler_params=pltpu.CompilerParams(needs_layout_passes=False),
        scratch_types=[
            pltpu.VMEM((2 * CH2,), jnp.int32),
            pltpu.VMEM((K * VP,), jnp.float32),
            pltpu.SemaphoreType.DMA,
            pltpu.SemaphoreType.DMA,
        ],
    )
    return kern(comb)


def _layer_norm(h, g, b):
    mu = jnp.mean(h, axis=-1, keepdims=True)
    var = jnp.mean((h - mu) ** 2, axis=-1, keepdims=True)
    return (h - mu) * lax.rsqrt(var + EPS) * g + b


RB = 512
NRB = VP // RB
PREC = lax.Precision.HIGHEST


def _fused_body(a_ref, x_ref, w_ref, b_ref, lg_ref, lb_ref, hw_ref, hb_ref,
                o_ref, abf_scr, h_scr, v_scr, dinv_scr, pool_scr):
    g = pl.program_id(0)
    l = pl.program_id(1)
    rb = pl.program_id(2)
    srow = g * VP + rb * RB
    lg = lg_ref[...]
    lb = lb_ref[...]

    @pl.when(l == 0)
    def _():
        a = a_ref[...]
        s = jnp.sum(a, axis=1, keepdims=True)
        dinv_scr[pl.ds(srow, RB), :] = lax.rsqrt(s + 1.0)
        abf_scr[pl.ds(rb * RB, RB), :] = a.astype(jnp.bfloat16)
        h_scr[pl.ds(srow, RB), :] = x_ref[pl.ds(rb * RB, RB), :]

    @pl.when((l > 0) & (rb == 0))
    def _():
        hg = h_scr[pl.ds(g * VP, VP), :]
        u = jnp.dot(hg, w_ref[0], preferred_element_type=jnp.float32,
                    precision=PREC)
        v_scr[...] = (dinv_scr[pl.ds(g * VP, VP), :] * u).astype(jnp.bfloat16)

    @pl.when(l > 0)
    def _():
        y = jnp.dot(abf_scr[pl.ds(rb * RB, RB), :], v_scr[...],
                    preferred_element_type=jnp.float32)
        y = y + v_scr[pl.ds(rb * RB, RB), :].astype(jnp.float32)
        h = dinv_scr[pl.ds(srow, RB), :] * y + b_ref[0, 0]
        h = jnp.maximum(h, 0.0)
        h = _layer_norm(h, lg, lb)
        h_scr[pl.ds(srow, RB), :] = h

        @pl.when(l == 3)
        def _():
            rowid = lax.broadcasted_iota(jnp.int32, (RB, 1), 0) + rb * RB
            hm = jnp.where(rowid < V, h, -1e30)
            m = jnp.max(hm, axis=0, keepdims=True)

            @pl.when(rb == 0)
            def _():
                pool_scr[pl.ds(g, 1), :] = m

            @pl.when(rb > 0)
            def _():
                pool_scr[pl.ds(g, 1), :] = jnp.maximum(
                    pool_scr[pl.ds(g, 1), :], m)

            @pl.when((g == N - 1) & (rb == NRB - 1))
            def _():
                pn = _layer_norm(pool_scr[...], lg, lb)
                o_ref[...] = jnp.dot(
                    pn, hw_ref[...], preferred_element_type=jnp.float32,
                    precision=PREC) + hb_ref[...]


def _run_dense(adj, x, W0, b0, W1, b1, W2, b2, ln_g, ln_b, head_W, head_b):
    xp = jnp.pad(x.reshape(N, V, D_IN),
                 ((0, 0), (0, VP - V), (0, DM - D_IN))).reshape(N * VP, DM)
    ws = jnp.stack([jnp.pad(W0, ((0, DM - D_IN), (0, 0))), W1, W2])
    bs = jnp.stack([b0, b1, b2]).reshape(3, 1, DM)

    full = lambda shape: pl.BlockSpec(shape, lambda g, l, r: (0,) * len(shape))
    return pl.pallas_call(
        _fused_body,
        grid=(N, 4, NRB),
        in_specs=[
            pl.BlockSpec((RB, VP),
                         lambda g, l, r: (g * NRB + jnp.where(l == 0, r, 0), 0)),
            pl.BlockSpec((VP, DM), lambda g, l, r: (g, 0)),
            pl.BlockSpec((1, DM, DM),
                         lambda g, l, r: (jnp.maximum(l, 1) - 1, 0, 0)),
            pl.BlockSpec((1, 1, DM),
                         lambda g, l, r: (jnp.maximum(l, 1) - 1, 0, 0)),
            full((DM,)), full((DM,)),
            full((DM, 3)), full((3,)),
        ],
        out_specs=pl.BlockSpec((N, 3), lambda g, l, r: (0, 0)),
        out_shape=jax.ShapeDtypeStruct((N, 3), jnp.float32),
        scratch_shapes=[
            pltpu.VMEM((VP, VP), jnp.bfloat16),
            pltpu.VMEM((N * VP, DM), jnp.float32),
            pltpu.VMEM((VP, DM), jnp.bfloat16),
            pltpu.VMEM((N * VP, 1), jnp.float32),
            pltpu.VMEM((N, DM), jnp.float32),
        ],
    )(adj, xp, ws, bs, ln_g, ln_b, head_W, head_b)


def kernel(x, edge_index, edge_weight, W0, b0, W1, b1, W2, b2,
           ln_g, ln_b, head_W, head_b):
    ei = edge_index.reshape(2, -1).astype(jnp.int32)
    flat_idx = ei[1] * VP + ei[0]
    ews = edge_weight.reshape(-1).astype(jnp.float32)
    comb = jnp.concatenate(
        [flat_idx.reshape(N * NCHPG, CHUNK),
         lax.bitcast_convert_type(ews, jnp.int32).reshape(N * NCHPG, CHUNK)],
        axis=1).reshape(-1)

    adj_flat = _build_dense_adj(comb)
    adj = adj_flat.reshape(N * VP, VP)

    return _run_dense(adj, x, W0, b0, W1, b1, W2, b2,
                      ln_g, ln_b, head_W, head_b)


# speedup vs baseline: 1.1719x; 1.1696x over previous
import functools

import jax
import jax.numpy as jnp
from jax import lax
from jax.experimental import pallas as pl
from jax.experimental.pallas import tpu as pltpu
from jax.experimental.pallas import tpu_sc as plsc

N = 4
V = 2500
VP = 2560
E = 80000
D_IN = 16
DM = 128
NW = 32
K = 40
PASSES = VP // (K * NW)
CHUNK = 4000
CH2 = 2 * CHUNK
NCHPG = E // CHUNK
EPS = 1e-5


def _adj_body(comb_hbm, out_hbm, buf, acc, shed, esem, osem):
    cid = lax.axis_index("c")
    sid = lax.axis_index("s")
    wid = sid * 2 + cid

    zero16 = jnp.zeros((16,), jnp.float32)

    def zero_body(i, _):
        acc[pl.ds(i * 16, 16)] = zero16
        return 0

    def edge_dma(ch, off):
        return pltpu.make_async_copy(shed.at[pl.ds(ch * CH2, CH2)],
                                     buf.at[pl.ds(off, CH2)], esem)

    slots = [(g, p) for g in range(N) for p in range(PASSES)]
    wo = None
    for s, (g, p) in enumerate(slots):
        c0 = (p * NW + wid) * (K * VP)

        if p == 0:
            plsc.subcore_barrier()

            @pl.when(sid == 0)
            def _():
                pltpu.sync_copy(
                    comb_hbm.at[pl.ds(g * NCHPG * CH2, NCHPG * CH2)], shed)

            plsc.subcore_barrier()

        edge_dma(0, 0).start()
        if wo is not None:
            wo.wait()
        lax.fori_loop(0, (K * VP) // 16, zero_body, 0)

        def chunk_body(ch, _, c0=c0):
            off = (ch % 2) * CH2
            edge_dma(ch, off).wait()

            @pl.when(ch < NCHPG - 1)
            def _():
                edge_dma(ch + 1, CH2 - off).start()

            @plsc.parallel_loop(0, CHUNK // 64, unroll=2)
            def edge_body(j):
                base = off + j * 64
                wbase = base + CHUNK
                for t in range(4):
                    i16 = buf[pl.ds(base + t * 16, 16)]
                    w16 = buf[pl.ds(wbase + t * 16, 16)]
                    rel = i16 - c0
                    m = plsc.bitcast(rel, jnp.uint32) < jnp.uint32(K * VP)
                    relc = jnp.where(m, rel, 0)
                    plsc.addupdate_scatter(acc, [relc],
                                           plsc.bitcast(w16, jnp.float32),
                                           mask=m)

            return 0

        lax.fori_loop(0, NCHPG, chunk_body, 0)

        wo = pltpu.make_async_copy(
            acc, out_hbm.at[pl.ds(g * VP * VP + c0, K * VP)], osem)
        wo.start()
    wo.wait()


def _build_dense_adj(comb):
    kern = pl.kernel(
        _adj_body,
        out_type=jax.ShapeDtypeStruct((N * VP * VP,), jnp.float32),
        mesh=plsc.VectorSubcoreMesh(core_axis_name="c", subcore_axis_name="s",
                                    num_cores=2, num_subcores=16),
        compiler_params=pltpu.CompilerParams(needs_layout_passes=False),
        scratch_types=[
            pltpu.VMEM((2 * CH2,), jnp.int32),
            pltpu.VMEM((K * VP,), jnp.float32),
            pltpu.VMEM_SHARED((NCHPG * CH2,), jnp.int32),
            pltpu.SemaphoreType.DMA,
            pltpu.SemaphoreType.DMA,
        ],
    )
    return kern(comb)


def _layer_norm(h, g, b):
    mu = jnp.mean(h, axis=-1, keepdims=True)
    var = jnp.mean((h - mu) ** 2, axis=-1, keepdims=True)
    return (h - mu) * lax.rsqrt(var + EPS) * g + b


RB = 512
NRB = VP // RB
PREC = lax.Precision.HIGHEST


def _fused_body(a_ref, x_ref, w_ref, b_ref, lg_ref, lb_ref, hw_ref, hb_ref,
                o_ref, abf_scr, h_scr, v_scr, dinv_scr, pool_scr):
    g = pl.program_id(0)
    l = pl.program_id(1)
    rb = pl.program_id(2)
    srow = g * VP + rb * RB
    lg = lg_ref[...]
    lb = lb_ref[...]

    @pl.when(l == 0)
    def _():
        a = a_ref[...]
        s = jnp.sum(a, axis=1, keepdims=True)
        dinv_scr[pl.ds(srow, RB), :] = lax.rsqrt(s + 1.0)
        abf_scr[pl.ds(rb * RB, RB), :] = a.astype(jnp.bfloat16)
        h_scr[pl.ds(srow, RB), :] = x_ref[pl.ds(rb * RB, RB), :]

    @pl.when((l > 0) & (rb == 0))
    def _():
        hg = h_scr[pl.ds(g * VP, VP), :]
        u = jnp.dot(hg, w_ref[0], preferred_element_type=jnp.float32,
                    precision=PREC)
        v_scr[...] = (dinv_scr[pl.ds(g * VP, VP), :] * u).astype(jnp.bfloat16)

    @pl.when(l > 0)
    def _():
        y = jnp.dot(abf_scr[pl.ds(rb * RB, RB), :], v_scr[...],
                    preferred_element_type=jnp.float32)
        y = y + v_scr[pl.ds(rb * RB, RB), :].astype(jnp.float32)
        h = dinv_scr[pl.ds(srow, RB), :] * y + b_ref[0, 0]
        h = jnp.maximum(h, 0.0)
        h = _layer_norm(h, lg, lb)
        h_scr[pl.ds(srow, RB), :] = h

        @pl.when(l == 3)
        def _():
            rowid = lax.broadcasted_iota(jnp.int32, (RB, 1), 0) + rb * RB
            hm = jnp.where(rowid < V, h, -1e30)
            m = jnp.max(hm, axis=0, keepdims=True)

            @pl.when(rb == 0)
            def _():
                pool_scr[pl.ds(g, 1), :] = m

            @pl.when(rb > 0)
            def _():
                pool_scr[pl.ds(g, 1), :] = jnp.maximum(
                    pool_scr[pl.ds(g, 1), :], m)

            @pl.when((g == N - 1) & (rb == NRB - 1))
            def _():
                pn = _layer_norm(pool_scr[...], lg, lb)
                o_ref[...] = jnp.dot(
                    pn, hw_ref[...], preferred_element_type=jnp.float32,
                    precision=PREC) + hb_ref[...]


def _run_dense(adj, x, W0, b0, W1, b1, W2, b2, ln_g, ln_b, head_W, head_b):
    xp = jnp.pad(x.reshape(N, V, D_IN),
                 ((0, 0), (0, VP - V), (0, DM - D_IN))).reshape(N * VP, DM)
    ws = jnp.stack([jnp.pad(W0, ((0, DM - D_IN), (0, 0))), W1, W2])
    bs = jnp.stack([b0, b1, b2]).reshape(3, 1, DM)

    full = lambda shape: pl.BlockSpec(shape, lambda g, l, r: (0,) * len(shape))
    return pl.pallas_call(
        _fused_body,
        grid=(N, 4, NRB),
        in_specs=[
            pl.BlockSpec((RB, VP),
                         lambda g, l, r: (g * NRB + jnp.where(l == 0, r, 0), 0)),
            pl.BlockSpec((VP, DM), lambda g, l, r: (g, 0)),
            pl.BlockSpec((1, DM, DM),
                         lambda g, l, r: (jnp.maximum(l, 1) - 1, 0, 0)),
            pl.BlockSpec((1, 1, DM),
                         lambda g, l, r: (jnp.maximum(l, 1) - 1, 0, 0)),
            full((DM,)), full((DM,)),
            full((DM, 3)), full((3,)),
        ],
        out_specs=pl.BlockSpec((N, 3), lambda g, l, r: (0, 0)),
        out_shape=jax.ShapeDtypeStruct((N, 3), jnp.float32),
        scratch_shapes=[
            pltpu.VMEM((VP, VP), jnp.bfloat16),
            pltpu.VMEM((N * VP, DM), jnp.float32),
            pltpu.VMEM((VP, DM), jnp.bfloat16),
            pltpu.VMEM((N * VP, 1), jnp.float32),
            pltpu.VMEM((N, DM), jnp.float32),
        ],
    )(adj, xp, ws, bs, ln_g, ln_b, head_W, head_b)


def kernel(x, edge_index, edge_weight, W0, b0, W1, b1, W2, b2,
           ln_g, ln_b, head_W, head_b):
    ei = edge_index.reshape(2, -1).astype(jnp.int32)
    flat_idx = ei[1] * VP + ei[0]
    ews = edge_weight.reshape(-1).astype(jnp.float32)
    comb = jnp.concatenate(
        [flat_idx.reshape(N * NCHPG, CHUNK),
         lax.bitcast_convert_type(ews, jnp.int32).reshape(N * NCHPG, CHUNK)],
        axis=1).reshape(-1)

    adj_flat = _build_dense_adj(comb)
    adj = adj_flat.reshape(N * VP, VP)

    return _run_dense(adj, x, W0, b0, W1, b1, W2, b2,
                      ln_g, ln_b, head_W, head_b)


# speedup vs baseline: 1.5285x; 1.3043x over previous
import functools

import jax
import jax.numpy as jnp
from jax import lax
from jax.experimental import pallas as pl
from jax.experimental.pallas import tpu as pltpu
from jax.experimental.pallas import tpu_sc as plsc

N = 4
V = 2500
VP = 2560
E = 80000
D_IN = 16
DM = 128
NW = 32
K = 40
PASSES = VP // (K * NW)
CHUNK = 4000
CH2 = 2 * CHUNK
NCHPG = E // CHUNK
EPS = 1e-5


def _adj_body(comb_hbm, out_hbm, buf, acc, shed, esem, osem):
    cid = lax.axis_index("c")
    sid = lax.axis_index("s")
    wid = sid * 2 + cid

    zero16 = jnp.zeros((16,), jnp.float32)

    def zero_body(i, _):
        acc[pl.ds(i * 16, 16)] = zero16
        return 0

    def edge_dma(ch, off):
        return pltpu.make_async_copy(shed.at[pl.ds(ch * CH2, CH2)],
                                     buf.at[pl.ds(off, CH2)], esem)

    @pl.when(sid == 0)
    def _():
        pltpu.sync_copy(comb_hbm, shed)

    plsc.subcore_barrier()

    wo = None
    for p in range(PASSES):
        c0 = (p * NW + wid) * (K * VP)

        edge_dma(0, 0).start()
        if wo is not None:
            wo.wait()
        lax.fori_loop(0, (K * VP) // 16, zero_body, 0)

        def chunk_body(ch, _, c0=c0):
            off = (ch % 2) * CH2
            edge_dma(ch, off).wait()

            @pl.when(ch < NCHPG - 1)
            def _():
                edge_dma(ch + 1, CH2 - off).start()

            @plsc.parallel_loop(0, CHUNK // 64, unroll=2)
            def edge_body(j):
                base = off + j * 64
                wbase = base + CHUNK
                for t in range(4):
                    i16 = buf[pl.ds(base + t * 16, 16)]
                    w16 = buf[pl.ds(wbase + t * 16, 16)]
                    rel = i16 - c0
                    m = plsc.bitcast(rel, jnp.uint32) < jnp.uint32(K * VP)
                    relc = jnp.where(m, rel, 0)
                    plsc.addupdate_scatter(acc, [relc],
                                           plsc.bitcast(w16, jnp.float32),
                                           mask=m)

            return 0

        lax.fori_loop(0, NCHPG, chunk_body, 0)

        wo = pltpu.make_async_copy(
            acc, out_hbm.at[pl.ds(c0, K * VP)], osem)
        wo.start()
    wo.wait()


def _build_dense_adj(comb_g):
    kern = pl.kernel(
        _adj_body,
        out_type=jax.ShapeDtypeStruct((VP * VP,), jnp.float32),
        mesh=plsc.VectorSubcoreMesh(core_axis_name="c", subcore_axis_name="s",
                                    num_cores=2, num_subcores=16),
        compiler_params=pltpu.CompilerParams(needs_layout_passes=False),
        scratch_types=[
            pltpu.VMEM((2 * CH2,), jnp.int32),
            pltpu.VMEM((K * VP,), jnp.float32),
            pltpu.VMEM_SHARED((NCHPG * CH2,), jnp.int32),
            pltpu.SemaphoreType.DMA,
            pltpu.SemaphoreType.DMA,
        ],
    )
    return kern(comb_g)


def _layer_norm(h, g, b):
    mu = jnp.mean(h, axis=-1, keepdims=True)
    var = jnp.mean((h - mu) ** 2, axis=-1, keepdims=True)
    return (h - mu) * lax.rsqrt(var + EPS) * g + b


RB = 512
NRB = VP // RB
PREC = lax.Precision.HIGHEST


def _fused_body(a_ref, x_ref, w_ref, b_ref, lg_ref, lb_ref,
                o_ref, abf_scr, h_scr, v_scr, dinv_scr):
    l = pl.program_id(0)
    rb = pl.program_id(1)
    srow = rb * RB
    lg = lg_ref[...]
    lb = lb_ref[...]

    @pl.when(l == 0)
    def _():
        a = a_ref[...]
        s = jnp.sum(a, axis=1, keepdims=True)
        dinv_scr[pl.ds(srow, RB), :] = lax.rsqrt(s + 1.0)
        abf_scr[pl.ds(srow, RB), :] = a.astype(jnp.bfloat16)
        h_scr[pl.ds(srow, RB), :] = x_ref[pl.ds(srow, RB), :]

    @pl.when((l > 0) & (rb == 0))
    def _():
        u = jnp.dot(h_scr[...], w_ref[0], preferred_element_type=jnp.float32,
                    precision=PREC)
        v_scr[...] = (dinv_scr[...] * u).astype(jnp.bfloat16)

    @pl.when(l > 0)
    def _():
        y = jnp.dot(abf_scr[pl.ds(srow, RB), :], v_scr[...],
                    preferred_element_type=jnp.float32)
        y = y + v_scr[pl.ds(srow, RB), :].astype(jnp.float32)
        h = dinv_scr[pl.ds(srow, RB), :] * y + b_ref[0, 0]
        h = jnp.maximum(h, 0.0)
        h = _layer_norm(h, lg, lb)
        h_scr[pl.ds(srow, RB), :] = h

        @pl.when(l == 3)
        def _():
            rowid = lax.broadcasted_iota(jnp.int32, (RB, 1), 0) + srow
            hm = jnp.where(rowid < V, h, -1e30)
            m = jnp.max(hm, axis=0, keepdims=True)[None]

            @pl.when(rb == 0)
            def _():
                o_ref[...] = m

            @pl.when(rb > 0)
            def _():
                o_ref[...] = jnp.maximum(o_ref[...], m)


def _head_body(p_ref, lg_ref, lb_ref, hw_ref, hb_ref, o_ref):
    pn = _layer_norm(p_ref[...], lg_ref[...], lb_ref[...])
    o_ref[...] = jnp.dot(pn, hw_ref[...], preferred_element_type=jnp.float32,
                         precision=PREC) + hb_ref[...]


def _run_dense_g(adj_g, xp_g, ws, bs, ln_g, ln_b):
    full = lambda shape: pl.BlockSpec(shape, lambda l, r: (0,) * len(shape))
    return pl.pallas_call(
        _fused_body,
        grid=(4, NRB),
        in_specs=[
            pl.BlockSpec((RB, VP),
                         lambda l, r: (jnp.where(l == 0, r, 0), 0)),
            full((VP, DM)),
            pl.BlockSpec((1, DM, DM),
                         lambda l, r: (jnp.maximum(l, 1) - 1, 0, 0)),
            pl.BlockSpec((1, 1, DM),
                         lambda l, r: (jnp.maximum(l, 1) - 1, 0, 0)),
            full((DM,)), full((DM,)),
        ],
        out_specs=pl.BlockSpec((1, 1, DM), lambda l, r: (0, 0, 0)),
        out_shape=jax.ShapeDtypeStruct((1, 1, DM), jnp.float32),
        scratch_shapes=[
            pltpu.VMEM((VP, VP), jnp.bfloat16),
            pltpu.VMEM((VP, DM), jnp.float32),
            pltpu.VMEM((VP, DM), jnp.bfloat16),
            pltpu.VMEM((VP, 1), jnp.float32),
        ],
    )(adj_g, xp_g, ws, bs, ln_g, ln_b)


def _run_dense(adjs, x, W0, b0, W1, b1, W2, b2, ln_g, ln_b, head_W, head_b):
    xp = jnp.pad(x.reshape(N, V, D_IN),
                 ((0, 0), (0, VP - V), (0, DM - D_IN))).reshape(N * VP, DM)
    ws = jnp.stack([jnp.pad(W0, ((0, DM - D_IN), (0, 0))), W1, W2])
    bs = jnp.stack([b0, b1, b2]).reshape(3, 1, DM)

    pooled = [
        _run_dense_g(adjs[g], lax.slice_in_dim(xp, g * VP, (g + 1) * VP),
                     ws, bs, ln_g, ln_b)
        for g in range(N)
    ]
    pooled = jnp.concatenate(pooled, axis=0).reshape(N, DM)
    return pl.pallas_call(
        _head_body,
        out_shape=jax.ShapeDtypeStruct((N, 3), jnp.float32),
    )(pooled, ln_g, ln_b, head_W, head_b)


def kernel(x, edge_index, edge_weight, W0, b0, W1, b1, W2, b2,
           ln_g, ln_b, head_W, head_b):
    ei = edge_index.reshape(2, -1).astype(jnp.int32)
    flat_idx = ei[1] * VP + ei[0]
    ews = edge_weight.reshape(-1).astype(jnp.float32)
    comb = jnp.concatenate(
        [flat_idx.reshape(N * NCHPG, CHUNK),
         lax.bitcast_convert_type(ews, jnp.int32).reshape(N * NCHPG, CHUNK)],
        axis=1).reshape(-1)

    adjs = [
        _build_dense_adj(
            lax.slice_in_dim(comb, g * NCHPG * CH2, (g + 1) * NCHPG * CH2)
        ).reshape(VP, VP)
        for g in range(N)
    ]

    return _run_dense(adjs, x, W0, b0, W1, b1, W2, b2,
                      ln_g, ln_b, head_W, head_b)
